# baseline (speedup 1.0000x reference)
import jax
import jax.numpy as jnp
from jax.experimental import pallas as pl
from jax.experimental.pallas import tpu as pltpu

NUM_EXPERTS = 8
TOP_K = 2
D_MODEL = 1024
D_FF = 2048
TOKEN_TILE = 256


def _moe_dense_kernel(logits_ref, x_ref, w1_ref, b1_ref, w2_ref, b2_ref, out_ref):
    e = pl.program_id(1)

    logits = logits_ref[...]
    tt = logits.shape[0]
    iota = jax.lax.broadcasted_iota(jnp.int32, (tt, NUM_EXPERTS), 1)
    m1 = jnp.max(logits, axis=-1, keepdims=True)
    a1 = jnp.argmax(logits, axis=-1)
    masked = jnp.where(iota == a1[:, None], -jnp.inf, logits)
    m2 = jnp.max(masked, axis=-1, keepdims=True)
    a2 = jnp.argmax(masked, axis=-1)
    w_first = 1.0 / (1.0 + jnp.exp(m2 - m1))
    w_second = 1.0 - w_first
    combine_e = (jnp.where(a1[:, None] == e, w_first, 0.0)
                 + jnp.where(a2[:, None] == e, w_second, 0.0))

    x = x_ref[...]
    h = jnp.maximum(
        jnp.dot(x, w1_ref[0], preferred_element_type=jnp.float32) + b1_ref[0, 0, :],
        0.0)
    y = jnp.dot(h, w2_ref[0], preferred_element_type=jnp.float32) + b2_ref[0, 0, :]

    contrib = combine_e * y

    @pl.when(e == 0)
    def _():
        out_ref[...] = contrib

    @pl.when(e != 0)
    def _():
        out_ref[...] += contrib


def kernel(hidden_states, router_logits, layer_idx, W1, b1, W2, b2):
    B, S, D = hidden_states.shape
    T = B * S
    E = NUM_EXPERTS
    x = hidden_states.reshape(T, D)
    logits = router_logits.reshape(T, E)

    nt = T // TOKEN_TILE
    out = pl.pallas_call(
        _moe_dense_kernel,
        grid=(nt, E),
        in_specs=[
            pl.BlockSpec((TOKEN_TILE, E), lambda t, e: (t, 0)),
            pl.BlockSpec((TOKEN_TILE, D), lambda t, e: (t, 0)),
            pl.BlockSpec((1, D, D_FF), lambda t, e: (e, 0, 0)),
            pl.BlockSpec((1, 1, D_FF), lambda t, e: (e, 0, 0)),
            pl.BlockSpec((1, D_FF, D), lambda t, e: (e, 0, 0)),
            pl.BlockSpec((1, 1, D), lambda t, e: (e, 0, 0)),
        ],
        out_specs=pl.BlockSpec((TOKEN_TILE, D), lambda t, e: (t, 0)),
        out_shape=jax.ShapeDtypeStruct((T, D), jnp.float32),
    )(logits, x, W1, b1.reshape(NUM_EXPERTS, 1, D_FF), W2,
      b2.reshape(NUM_EXPERTS, 1, D))
    return out.reshape(B, S, D)


# speedup vs baseline: 1.5660x; 1.5660x over previous
import jax
import jax.numpy as jnp
from jax.experimental import pallas as pl
from jax.experimental.pallas import tpu as pltpu

NUM_EXPERTS = 8
TOP_K = 2
D_MODEL = 1024
D_FF = 2048
TOKEN_TILE = 256


def _moe_dense_kernel(logits_ref, x_ref, w1_ref, b1_ref, w2_ref, b2_ref, out_ref):
    e = pl.program_id(0)
    t = pl.program_id(1)

    logits = logits_ref[...]
    tt = logits.shape[0]
    iota = jax.lax.broadcasted_iota(jnp.int32, (tt, NUM_EXPERTS), 1)
    m1 = jnp.max(logits, axis=-1, keepdims=True)
    a1 = jnp.argmax(logits, axis=-1)
    masked = jnp.where(iota == a1[:, None], -jnp.inf, logits)
    m2 = jnp.max(masked, axis=-1, keepdims=True)
    a2 = jnp.argmax(masked, axis=-1)
    w_first = 1.0 / (1.0 + jnp.exp(m2 - m1))
    w_second = 1.0 - w_first
    combine_e = (jnp.where(a1[:, None] == e, w_first, 0.0)
                 + jnp.where(a2[:, None] == e, w_second, 0.0))

    x = x_ref[...]
    h = jnp.maximum(
        jnp.dot(x, w1_ref[0], preferred_element_type=jnp.float32) + b1_ref[0, 0, :],
        0.0)
    y = jnp.dot(h, w2_ref[0], preferred_element_type=jnp.float32) + b2_ref[0, 0, :]

    contrib = combine_e * y
    sl = pl.ds(t * TOKEN_TILE, TOKEN_TILE)

    @pl.when(e == 0)
    def _():
        out_ref[sl, :] = contrib

    @pl.when(e != 0)
    def _():
        out_ref[sl, :] += contrib


def kernel(hidden_states, router_logits, layer_idx, W1, b1, W2, b2):
    B, S, D = hidden_states.shape
    T = B * S
    E = NUM_EXPERTS
    x = hidden_states.reshape(T, D)
    logits = router_logits.reshape(T, E)

    nt = T // TOKEN_TILE
    out = pl.pallas_call(
        _moe_dense_kernel,
        grid=(E, nt),
        in_specs=[
            pl.BlockSpec((TOKEN_TILE, E), lambda e, t: (t, 0)),
            pl.BlockSpec((TOKEN_TILE, D), lambda e, t: (t, 0)),
            pl.BlockSpec((1, D, D_FF), lambda e, t: (e, 0, 0)),
            pl.BlockSpec((1, 1, D_FF), lambda e, t: (e, 0, 0)),
            pl.BlockSpec((1, D_FF, D), lambda e, t: (e, 0, 0)),
            pl.BlockSpec((1, 1, D), lambda e, t: (e, 0, 0)),
        ],
        out_specs=pl.BlockSpec((T, D), lambda e, t: (0, 0)),
        out_shape=jax.ShapeDtypeStruct((T, D), jnp.float32),
    )(logits, x, W1, b1.reshape(NUM_EXPERTS, 1, D_FF), W2,
      b2.reshape(NUM_EXPERTS, 1, D))
    return out.reshape(B, S, D)


# speedup vs baseline: 1.5816x; 1.0099x over previous
import functools

import jax
import jax.numpy as jnp
from jax import lax
from jax.experimental import pallas as pl
from jax.experimental.pallas import tpu as pltpu
from jax.experimental.pallas import tpu_sc as plsc

NUM_EXPERTS = 8
D_MODEL = 1024
D_FF = 2048
T_TOKENS = 2048
ROW_TILE = 128
NUM_PAIRS = 2 * T_TOKENS
NROWS = NUM_PAIRS + NUM_EXPERTS * ROW_TILE
NT = NROWS // ROW_TILE

NC = 2
NS = 16
NW = NC * NS
TOK_PER_W = T_TOKENS // NW


def _route_math(logits):
    T, E = logits.shape
    iota_e = lax.broadcasted_iota(jnp.int32, (T, E), 1)
    m1 = jnp.max(logits, axis=-1, keepdims=True)
    oh1_b = logits == m1
    first1 = jnp.min(jnp.where(oh1_b, iota_e, E), axis=-1, keepdims=True)
    oh1 = (iota_e == first1).astype(jnp.float32)
    masked = jnp.where(iota_e == first1, -jnp.inf, logits)
    m2 = jnp.max(masked, axis=-1, keepdims=True)
    oh2_b = masked == m2
    first2 = jnp.min(jnp.where(oh2_b, iota_e, E), axis=-1, keepdims=True)
    oh2 = (iota_e == first2).astype(jnp.float32)

    w_first = 1.0 / (1.0 + jnp.exp(m2 - m1))
    w_second = 1.0 - w_first

    oh = jnp.concatenate([oh1, oh2], axis=0)
    c = oh
    s = 1
    while s < 2 * T:
        c = c + jnp.concatenate(
            [jnp.zeros((s, E), jnp.float32), c[: 2 * T - s]], axis=0)
        s *= 2
    counts = c[2 * T - 1:2 * T, :]
    excl = c - oh
    rank = jnp.sum(oh * excl, axis=-1)

    pc = jnp.floor((counts + (ROW_TILE - 1)) / ROW_TILE) * ROW_TILE
    co = lax.broadcasted_iota(jnp.int32, (E, E), 1)
    ro = lax.broadcasted_iota(jnp.int32, (E, E), 0)
    po = jnp.sum(jnp.where(co < ro, jnp.broadcast_to(pc, (E, E)), 0.0),
                 axis=-1)
    po_row = po.reshape(1, E)
    dst = (jnp.sum(oh * jnp.broadcast_to(po_row, (2 * T, E)), axis=-1)
           + rank).astype(jnp.int32)

    ce = po_row + pc
    ti = lax.broadcasted_iota(jnp.int32, (NT, E), 0) * ROW_TILE
    te = jnp.sum((ti.astype(jnp.float32)
                  >= jnp.broadcast_to(ce, (NT, E))).astype(jnp.int32),
                 axis=-1)
    te = jnp.minimum(te, NUM_EXPERTS - 1)
    w = jnp.concatenate([w_first, w_second], axis=0).reshape(2, T)
    return dst.reshape(2, T), w, te.reshape(1, NT)


def _route_kernel(logits_ref, dst_ref, w_ref, te_ref):
    dst, w, te = _route_math(logits_ref[...])
    dst_ref[...] = dst
    w_ref[...] = w
    te_ref[...] = te


def _route(logits):
    return pl.pallas_call(
        _route_kernel,
        out_shape=[
            jax.ShapeDtypeStruct((2, T_TOKENS), jnp.int32),
            jax.ShapeDtypeStruct((2, T_TOKENS), jnp.float32),
            jax.ShapeDtypeStruct((1, NT), jnp.int32),
        ],
    )(logits)


PAIRS_PER_W = NUM_PAIRS // NW


def _dispatch_kernel(x_hbm, dst_hbm, dstf_hbm, wf_hbm, xs_hbm, rw_hbm,
                     idx_v, rows_v, dvf_v, wvf_v, sem):
    wid = lax.axis_index("s") * NC + lax.axis_index("c")
    base = wid * TOK_PER_W
    pltpu.sync_copy(x_hbm.at[pl.ds(base, TOK_PER_W)], rows_v)
    for j in range(2):
        pltpu.sync_copy(dst_hbm.at[j, pl.ds(base, TOK_PER_W)], idx_v)
        pltpu.async_copy(rows_v, xs_hbm.at[idx_v], sem).wait()

    pbase = wid * PAIRS_PER_W
    pltpu.sync_copy(dstf_hbm.at[pl.ds(pbase, PAIRS_PER_W)], dvf_v)
    pltpu.sync_copy(wf_hbm.at[pl.ds(pbase, PAIRS_PER_W)], wvf_v)
    pltpu.async_copy(wvf_v, rw_hbm.at[dvf_v], sem).wait()


def _dispatch(x, dst, w):
    mesh = plsc.VectorSubcoreMesh(core_axis_name="c", subcore_axis_name="s")
    return pl.kernel(
        _dispatch_kernel,
        mesh=mesh,
        out_type=[
            jax.ShapeDtypeStruct((NROWS, D_MODEL), jnp.float32),
            jax.ShapeDtypeStruct((NROWS,), jnp.float32),
        ],
        scratch_types=[
            pltpu.VMEM((TOK_PER_W,), jnp.int32),
            pltpu.VMEM((TOK_PER_W, D_MODEL), jnp.float32),
            pltpu.VMEM((PAIRS_PER_W,), jnp.int32),
            pltpu.VMEM((PAIRS_PER_W,), jnp.float32),
            pltpu.SemaphoreType.DMA,
        ],
    )(x, dst, dst.reshape(NUM_PAIRS), w.reshape(NUM_PAIRS))


def _ffn_kernel(te_ref, rw_ref, xs_ref, w1_ref, b1_ref, w2_ref,
                b2_ref, ys_ref):
    x = xs_ref[...]
    h = jnp.maximum(
        jnp.dot(x, w1_ref[0], preferred_element_type=jnp.float32)
        + b1_ref[0, 0, :], 0.0)
    y = jnp.dot(h, w2_ref[0], preferred_element_type=jnp.float32) + b2_ref[0, 0, :]
    ys_ref[...] = y * rw_ref[...]


def _ffn(te, rw, xs, W1, b1, W2, b2):
    grid_spec = pltpu.PrefetchScalarGridSpec(
        num_scalar_prefetch=1,
        grid=(NT,),
        in_specs=[
            pl.BlockSpec((ROW_TILE, 1), lambda i, te: (i, 0)),
            pl.BlockSpec((ROW_TILE, D_MODEL), lambda i, te: (i, 0)),
            pl.BlockSpec((1, D_MODEL, D_FF), lambda i, te: (te[0, i], 0, 0)),
            pl.BlockSpec((1, 1, D_FF), lambda i, te: (te[0, i], 0, 0)),
            pl.BlockSpec((1, D_FF, D_MODEL), lambda i, te: (te[0, i], 0, 0)),
            pl.BlockSpec((1, 1, D_MODEL), lambda i, te: (te[0, i], 0, 0)),
        ],
        out_specs=pl.BlockSpec((ROW_TILE, D_MODEL), lambda i, te: (i, 0)),
    )
    return pl.pallas_call(
        _ffn_kernel,
        grid_spec=grid_spec,
        out_shape=jax.ShapeDtypeStruct((NROWS, D_MODEL), jnp.float32),
    )(te, rw.reshape(NROWS, 1), xs,
      W1, b1.reshape(NUM_EXPERTS, 1, D_FF), W2, b2.reshape(NUM_EXPERTS, 1, D_MODEL))


CHUNK = 32


def _combine_kernel(ys_hbm, dst_hbm, out_hbm, idx0_v, idx1_v, y0_v, y1_v, sem0, sem1):
    wid = lax.axis_index("s") * NC + lax.axis_index("c")
    base = wid * TOK_PER_W
    for cidx in range(TOK_PER_W // CHUNK):
        tb = base + cidx * CHUNK
        pltpu.sync_copy(dst_hbm.at[0, pl.ds(tb, CHUNK)], idx0_v)
        pltpu.sync_copy(dst_hbm.at[1, pl.ds(tb, CHUNK)], idx1_v)
        cp0 = pltpu.async_copy(ys_hbm.at[idx0_v], y0_v, sem0)
        cp1 = pltpu.async_copy(ys_hbm.at[idx1_v], y1_v, sem1)
        cp0.wait()
        cp1.wait()

        def body(r, _):
            def inner(g, _):
                sl = pl.ds(g * 16, 16)
                y0_v[r, sl] = y0_v[r, sl] + y1_v[r, sl]
                return 0
            return lax.fori_loop(0, D_MODEL // 16, inner, 0)

        lax.fori_loop(0, CHUNK, body, 0)
        pltpu.sync_copy(y0_v, out_hbm.at[pl.ds(tb, CHUNK)])


def _combine(ys, dst):
    mesh = plsc.VectorSubcoreMesh(core_axis_name="c", subcore_axis_name="s")
    return pl.kernel(
        _combine_kernel,
        mesh=mesh,
        out_type=jax.ShapeDtypeStruct((T_TOKENS, D_MODEL), jnp.float32),
        scratch_types=[
            pltpu.VMEM((CHUNK,), jnp.int32),
            pltpu.VMEM((CHUNK,), jnp.int32),
            pltpu.VMEM((CHUNK, D_MODEL), jnp.float32),
            pltpu.VMEM((CHUNK, D_MODEL), jnp.float32),
            pltpu.SemaphoreType.DMA,
            pltpu.SemaphoreType.DMA,
        ],
    )(ys, dst)


def kernel(hidden_states, router_logits, layer_idx, W1, b1, W2, b2):
    B, S, D = hidden_states.shape
    x = hidden_states.reshape(B * S, D)
    logits = router_logits.reshape(B * S, NUM_EXPERTS)

    dst, w, te = _route(logits)
    xs, rw = _dispatch(x, dst, w)
    ys = _ffn(te, rw, xs, W1, b1, W2, b2)
    out = _combine(ys, dst)
    return out.reshape(B, S, D)
